# baseline (speedup 1.0000x reference)
import functools

import jax
import jax.numpy as jnp
from jax import lax
from jax.experimental import pallas as pl
from jax.experimental.pallas import tpu as pltpu
from jax.experimental.pallas import tpu_sc as plsc

N = 10000
E = 320000
D = 128
NC = 2
NS = 16
NW = NC * NS
K = 64
NSLOT = 4
NB = 16
B = 10
CT = NB * B
ET = CT * K
EPAD = ET * NW
NPAD = 10240
RPT = NPAD // NS


@functools.cache
def _mesh():
    return plsc.VectorSubcoreMesh(core_axis_name="c", subcore_axis_name="s",
                                  num_cores=NC, num_subcores=NS)


def _sc_out_type(want_cnt):
    out = [
        jax.ShapeDtypeStruct((NPAD, D), jnp.float32),
        jax.ShapeDtypeStruct((NPAD, D), jnp.float32),
    ]
    if want_cnt:
        out += [
            jax.ShapeDtypeStruct((NPAD,), jnp.float32),
            jax.ShapeDtypeStruct((NPAD,), jnp.float32),
        ]
    return out


def _sc_scratch(want_cnt):
    scratch = [
        pltpu.VMEM_SHARED((NPAD, D), jnp.float32),
        pltpu.VMEM((2, B, K), jnp.int32),
        pltpu.VMEM((2, B, K), jnp.int32),
        pltpu.SemaphoreType.DMA,
        pltpu.SemaphoreType.DMA,
    ]
    scratch += [pltpu.VMEM((K, D), jnp.float32) for _ in range(NSLOT)]
    scratch += [pltpu.SemaphoreType.DMA for _ in range(2 * NSLOT)]
    if want_cnt:
        scratch += [
            pltpu.VMEM_SHARED((NPAD,), jnp.float32),
            pltpu.VMEM((K * 2,), jnp.float32),
            pltpu.VMEM((K,), jnp.float32),
        ]
        scratch += [pltpu.SemaphoreType.DMA for _ in range(NSLOT)]
    return scratch


def _make_sc_body(want_cnt):
    def body(x_hbm, sd_hbm, *rest):
        if want_cnt:
            (acc0, acc1, cnt0, cnt1, acc_sh, sdb0, sdb1, semd0, semd1,
             *rest2) = rest
            rows = rest2[:NSLOT]
            semr = rest2[NSLOT:2 * NSLOT]
            sems = rest2[2 * NSLOT:3 * NSLOT]
            cnt_sh, zcnt, ones_v = rest2[3 * NSLOT:3 * NSLOT + 3]
            semc = rest2[3 * NSLOT + 3:]
        else:
            (acc0, acc1, acc_sh, sdb0, sdb1, semd0, semd1, *rest2) = rest
            rows = rest2[:NSLOT]
            semr = rest2[NSLOT:2 * NSLOT]
            sems = rest2[2 * NSLOT:3 * NSLOT]
            cnt_sh = zcnt = ones_v = semc = None
        rows0 = rows[0]
        sdb = (sdb0, sdb1)
        semd = (semd0, semd1)
        c = lax.axis_index("c")
        s = lax.axis_index("s")
        wid = c * NS + s

        def zrow(r, carry):
            for cc in range(D // 16):
                rows0[r, pl.ds(cc * 16, 16)] = jnp.zeros((16,), jnp.float32)
            return carry
        lax.fori_loop(0, K, zrow, 0)
        for j in range(RPT // K):
            pltpu.sync_copy(rows0, acc_sh.at[pl.ds(s * RPT + j * K, K)])
        if want_cnt:
            def zfill(r, carry):
                zcnt[pl.ds(r * 16, 16)] = jnp.zeros((16,), jnp.float32)
                return carry
            lax.fori_loop(0, 2 * K // 16, zfill, 0)
            for cc in range(K // 16):
                ones_v[pl.ds(cc * 16, 16)] = jnp.ones((16,), jnp.float32)
            for j in range(RPT // (2 * K)):
                pltpu.sync_copy(zcnt,
                                cnt_sh.at[pl.ds(s * RPT + j * 2 * K, 2 * K)])
        plsc.subcore_barrier()

        def gwait(b):
            pltpu.make_async_copy(x_hbm.at[sdb0.at[0, 0]], rows[b],
                                  semr[b]).wait()

        def swait(b):
            pltpu.make_async_copy(rows[b], acc_sh.at[sdb0.at[1, 0]],
                                  sems[b]).wait()
            if want_cnt:
                pltpu.make_async_copy(ones_v, cnt_sh.at[sdb0.at[1, 0]],
                                      semc[b]).wait()

        def dwait(sl):
            pltpu.make_async_copy(sd_hbm.at[wid, 0], sdb[sl],
                                  semd[sl]).wait()

        pltpu.sync_copy(sd_hbm.at[wid, 0], sdb0)
        pltpu.async_copy(sd_hbm.at[wid, 1], sdb1, semd1)
        for i in range(NSLOT - 1):
            pltpu.async_copy(x_hbm.at[sdb0.at[0, i]], rows[i], semr[i])

        for i in range(CT):
            b = i % NSLOT
            cur = sdb[(i // B) % 2]
            gwait(b)
            pltpu.make_async_copy(
                rows[b], acc_sh.at[cur.at[1, i % B]], sems[b]).start(add=True)
            if want_cnt:
                pltpu.make_async_copy(
                    ones_v, cnt_sh.at[cur.at[1, i % B]], semc[b]).start(
                        add=True)
            ni = i + NSLOT - 1
            if ni < CT:
                bn = ni % NSLOT
                if i >= 1:
                    swait(bn)
                if i % B == 0 and B <= i < (NB - 1) * B:
                    sl = (i // B + 1) % 2
                    pltpu.async_copy(sd_hbm.at[wid, i // B + 1], sdb[sl],
                                     semd[sl])
                if ni % B == 0:
                    dwait((ni // B) % 2)
                nxt = sdb[(ni // B) % 2]
                pltpu.async_copy(x_hbm.at[nxt.at[0, ni % B]], rows[bn],
                                 semr[bn])
        for b in range(NSLOT):
            swait(b)

        plsc.subcore_barrier()

        rb = s * RPT

        @pl.when(c == 0)
        def _():
            pltpu.sync_copy(acc_sh.at[pl.ds(rb, RPT)], acc0.at[pl.ds(rb, RPT)])
            if want_cnt:
                pltpu.sync_copy(cnt_sh.at[pl.ds(rb, RPT)],
                                cnt0.at[pl.ds(rb, RPT)])

        @pl.when(c == 1)
        def _():
            pltpu.sync_copy(acc_sh.at[pl.ds(rb, RPT)], acc1.at[pl.ds(rb, RPT)])
            if want_cnt:
                pltpu.sync_copy(cnt_sh.at[pl.ds(rb, RPT)],
                                cnt1.at[pl.ds(rb, RPT)])

    return body


@functools.cache
def _sc_agg_kernel(want_cnt):
    return pl.kernel(
        _make_sc_body(want_cnt),
        out_type=_sc_out_type(want_cnt),
        mesh=_mesh(),
        scratch_types=_sc_scratch(want_cnt),
    )


def _tc_pre_body(x_ref, wr_ref, bl_ref, o_ref):
    o_ref[...] = jnp.dot(x_ref[...], wr_ref[...],
                         preferred_element_type=jnp.float32) + bl_ref[...]


def _tc_pre(x, wrt, bl):
    rows = x.shape[0]
    block_rows = 2048 if rows % 2048 == 0 else 2000
    row_spec = pl.BlockSpec((block_rows, D), lambda i: (i, 0))
    return pl.pallas_call(
        _tc_pre_body,
        grid=(rows // block_rows,),
        in_specs=[row_spec, pl.BlockSpec((D, D), lambda i: (0, 0)),
                  pl.BlockSpec((1, D), lambda i: (0, 0))],
        out_specs=row_spec,
        out_shape=jax.ShapeDtypeStruct((rows, D), jnp.float32),
    )(x, wrt, bl)


def _tc_post_body(relu, a0_ref, a1_ref, c0_ref, c1_ref, xw_ref, wl_ref,
                  o_ref):
    agg = a0_ref[...] + a1_ref[...]
    inv = 1.0 / jnp.maximum(c0_ref[...] + c1_ref[...], 1.0)
    z = (jnp.dot(agg * inv, wl_ref[...], preferred_element_type=jnp.float32)
         + xw_ref[...])
    o_ref[...] = jnp.maximum(z, 0.0) if relu else z


def _tc_post(relu, out_rows, block_rows, acc0, acc1, cnt0, cnt1, xw, wlt):
    grid = out_rows // block_rows
    row_spec = pl.BlockSpec((block_rows, D), lambda i: (i, 0))
    cnt_spec = pl.BlockSpec((block_rows, 1), lambda i: (i, 0))
    full = pl.BlockSpec((D, D), lambda i: (0, 0))
    return pl.pallas_call(
        functools.partial(_tc_post_body, relu),
        grid=(grid,),
        in_specs=[row_spec, row_spec, cnt_spec, cnt_spec, row_spec, full],
        out_specs=row_spec,
        out_shape=jax.ShapeDtypeStruct((out_rows, D), jnp.float32),
    )(acc0, acc1, cnt0, cnt1, xw, wlt)


def kernel(x, edge_index, Wl1, bl1, Wr1, Wl2, bl2, Wr2):
    src = edge_index[0].astype(jnp.int32)
    dst = edge_index[1].astype(jnp.int32)
    pad = EPAD - E
    trash = N + jnp.arange(pad, dtype=jnp.int32) % (NPAD - N)
    src_pad = jnp.concatenate([src, trash])
    dst_pad = jnp.concatenate([dst, trash])
    src_rr = src_pad.reshape(CT, NW, K).transpose(1, 0, 2)
    dst_rr = dst_pad.reshape(CT, NW, K).transpose(1, 0, 2)
    sd = jnp.stack([src_rr.reshape(NW, NB, B, K),
                    dst_rr.reshape(NW, NB, B, K)], axis=2)
    x_pad = jnp.concatenate([x, jnp.zeros((NPAD - N, D), x.dtype)])

    xw1 = _tc_pre(x_pad, Wr1.T, bl1.reshape(1, D))
    acc0, acc1, cnt0, cnt1 = _sc_agg_kernel(True)(x_pad, sd)
    cnt0 = cnt0.reshape(NPAD, 1)
    cnt1 = cnt1.reshape(NPAD, 1)

    h_pad = _tc_post(True, NPAD, 2048, acc0, acc1, cnt0, cnt1, xw1, Wl1.T)

    hw2 = _tc_pre(h_pad, Wr2.T, bl2.reshape(1, D))
    b0, b1 = _sc_agg_kernel(False)(h_pad, sd)

    out = _tc_post(False, N, 2000, b0, b1, cnt0, cnt1, hw2, Wl2.T)
    return out


# speedup vs baseline: 1.0372x; 1.0372x over previous
import functools

import jax
import jax.numpy as jnp
from jax import lax
from jax.experimental import pallas as pl
from jax.experimental.pallas import tpu as pltpu
from jax.experimental.pallas import tpu_sc as plsc

N = 10000
E = 320000
D = 128
NC = 2
NS = 16
NW = NC * NS
K = 64
NSLOT = 4
NB = 16
B = 10
CT = NB * B
ET = CT * K
EPAD = ET * NW
NPAD = 10240
RPT = NPAD // NS


@functools.cache
def _mesh():
    return plsc.VectorSubcoreMesh(core_axis_name="c", subcore_axis_name="s",
                                  num_cores=NC, num_subcores=NS)


def _sc_out_type(want_cnt):
    out = [
        jax.ShapeDtypeStruct((NPAD, D), jnp.float32),
        jax.ShapeDtypeStruct((NPAD, D), jnp.float32),
    ]
    if want_cnt:
        out += [
            jax.ShapeDtypeStruct((NPAD,), jnp.float32),
            jax.ShapeDtypeStruct((NPAD,), jnp.float32),
        ]
    return out


def _sc_scratch(want_cnt):
    scratch = [
        pltpu.VMEM_SHARED((NPAD, D), jnp.float32),
        pltpu.VMEM((B * K,), jnp.int32),
        pltpu.VMEM((B * K,), jnp.int32),
        pltpu.VMEM((B, K), jnp.int32),
        pltpu.VMEM((B, K), jnp.int32),
        pltpu.SemaphoreType.DMA,
        pltpu.SemaphoreType.DMA,
        pltpu.SemaphoreType.DMA,
        pltpu.SemaphoreType.DMA,
    ]
    scratch += [pltpu.VMEM((K, D), jnp.float32) for _ in range(NSLOT)]
    scratch += [pltpu.SemaphoreType.DMA for _ in range(2 * NSLOT)]
    if want_cnt:
        scratch += [
            pltpu.VMEM_SHARED((NPAD,), jnp.float32),
            pltpu.VMEM((K * 2,), jnp.float32),
            pltpu.VMEM((K,), jnp.float32),
        ]
        scratch += [pltpu.SemaphoreType.DMA for _ in range(NSLOT)]
    return scratch


def _make_sc_body(want_cnt):
    def body(x_hbm, src_hbm, dst_hbm, *rest):
        if want_cnt:
            (acc0, acc1, cnt0, cnt1, acc_sh, sbuf0, sbuf1, dbuf0, dbuf1,
             semds0, semds1, semdd0, semdd1, *rest2) = rest
            rows = rest2[:NSLOT]
            semr = rest2[NSLOT:2 * NSLOT]
            sems = rest2[2 * NSLOT:3 * NSLOT]
            cnt_sh, zcnt, ones_v = rest2[3 * NSLOT:3 * NSLOT + 3]
            semc = rest2[3 * NSLOT + 3:]
        else:
            (acc0, acc1, acc_sh, sbuf0, sbuf1, dbuf0, dbuf1,
             semds0, semds1, semdd0, semdd1, *rest2) = rest
            rows = rest2[:NSLOT]
            semr = rest2[NSLOT:2 * NSLOT]
            sems = rest2[2 * NSLOT:3 * NSLOT]
            cnt_sh = zcnt = ones_v = semc = None
        rows0 = rows[0]
        sbuf = (sbuf0, sbuf1)
        dbuf = (dbuf0, dbuf1)
        semds = (semds0, semds1)
        semdd = (semdd0, semdd1)
        c = lax.axis_index("c")
        s = lax.axis_index("s")
        wid = c * NS + s

        def zrow(r, carry):
            for cc in range(D // 16):
                rows0[r, pl.ds(cc * 16, 16)] = jnp.zeros((16,), jnp.float32)
            return carry
        lax.fori_loop(0, K, zrow, 0)
        for j in range(RPT // K):
            pltpu.sync_copy(rows0, acc_sh.at[pl.ds(s * RPT + j * K, K)])
        if want_cnt:
            def zfill(r, carry):
                zcnt[pl.ds(r * 16, 16)] = jnp.zeros((16,), jnp.float32)
                return carry
            lax.fori_loop(0, 2 * K // 16, zfill, 0)
            for cc in range(K // 16):
                ones_v[pl.ds(cc * 16, 16)] = jnp.ones((16,), jnp.float32)
            for j in range(RPT // (2 * K)):
                pltpu.sync_copy(zcnt,
                                cnt_sh.at[pl.ds(s * RPT + j * 2 * K, 2 * K)])
        plsc.subcore_barrier()

        ebase = wid * ET

        def gwait(b):
            pltpu.make_async_copy(x_hbm.at[sbuf0.at[pl.ds(0, K)]], rows[b],
                                  semr[b]).wait()

        def swait(b):
            pltpu.make_async_copy(rows[b], acc_sh.at[dbuf0.at[0]],
                                  sems[b]).wait()
            if want_cnt:
                pltpu.make_async_copy(ones_v, cnt_sh.at[dbuf0.at[0]],
                                      semc[b]).wait()

        def dload(nb, sl, sem_s, sem_d):
            pltpu.async_copy(src_hbm.at[pl.ds(ebase + nb * B * K, B * K)],
                             sbuf[sl], sem_s[sl])
            pltpu.async_copy(dst_hbm.at[wid, nb], dbuf[sl], sem_d[sl])

        def dwait(sl):
            pltpu.make_async_copy(src_hbm.at[pl.ds(0, B * K)], sbuf[sl],
                                  semds[sl]).wait()
            pltpu.make_async_copy(dst_hbm.at[wid, 0], dbuf[sl],
                                  semdd[sl]).wait()

        pltpu.sync_copy(src_hbm.at[pl.ds(ebase, B * K)], sbuf0)
        pltpu.sync_copy(dst_hbm.at[wid, 0], dbuf0)
        dload(1, 1, semds, semdd)
        for i in range(NSLOT - 1):
            pltpu.async_copy(x_hbm.at[sbuf0.at[pl.ds(i * K, K)]], rows[i],
                             semr[i])

        for i in range(CT):
            b = i % NSLOT
            cur = dbuf[(i // B) % 2]
            gwait(b)
            pltpu.make_async_copy(
                rows[b], acc_sh.at[cur.at[i % B]], sems[b]).start(add=True)
            if want_cnt:
                pltpu.make_async_copy(
                    ones_v, cnt_sh.at[cur.at[i % B]], semc[b]).start(
                        add=True)
            ni = i + NSLOT - 1
            if ni < CT:
                bn = ni % NSLOT
                if i >= 1:
                    swait(bn)
                if i % B == 0 and B <= i < (NB - 1) * B:
                    dload(i // B + 1, (i // B + 1) % 2, semds, semdd)
                if ni % B == 0:
                    dwait((ni // B) % 2)
                nxt = sbuf[(ni // B) % 2]
                pltpu.async_copy(
                    x_hbm.at[nxt.at[pl.ds((ni % B) * K, K)]], rows[bn],
                    semr[bn])
        for b in range(NSLOT):
            swait(b)

        plsc.subcore_barrier()

        rb = s * RPT

        @pl.when(c == 0)
        def _():
            pltpu.sync_copy(acc_sh.at[pl.ds(rb, RPT)], acc0.at[pl.ds(rb, RPT)])
            if want_cnt:
                pltpu.sync_copy(cnt_sh.at[pl.ds(rb, RPT)],
                                cnt0.at[pl.ds(rb, RPT)])

        @pl.when(c == 1)
        def _():
            pltpu.sync_copy(acc_sh.at[pl.ds(rb, RPT)], acc1.at[pl.ds(rb, RPT)])
            if want_cnt:
                pltpu.sync_copy(cnt_sh.at[pl.ds(rb, RPT)],
                                cnt1.at[pl.ds(rb, RPT)])

    return body


@functools.cache
def _sc_agg_kernel(want_cnt):
    return pl.kernel(
        _make_sc_body(want_cnt),
        out_type=_sc_out_type(want_cnt),
        mesh=_mesh(),
        scratch_types=_sc_scratch(want_cnt),
    )


def _tc_pre_body(x_ref, wr_ref, bl_ref, o_ref):
    o_ref[...] = jnp.dot(x_ref[...], wr_ref[...],
                         preferred_element_type=jnp.float32) + bl_ref[...]


def _tc_pre(x, wrt, bl):
    rows = x.shape[0]
    block_rows = 2048 if rows % 2048 == 0 else 2000
    row_spec = pl.BlockSpec((block_rows, D), lambda i: (i, 0))
    return pl.pallas_call(
        _tc_pre_body,
        grid=(rows // block_rows,),
        in_specs=[row_spec, pl.BlockSpec((D, D), lambda i: (0, 0)),
                  pl.BlockSpec((1, D), lambda i: (0, 0))],
        out_specs=row_spec,
        out_shape=jax.ShapeDtypeStruct((rows, D), jnp.float32),
    )(x, wrt, bl)


def _tc_post_body(relu, a0_ref, a1_ref, c0_ref, c1_ref, xw_ref, wl_ref,
                  o_ref):
    agg = a0_ref[...] + a1_ref[...]
    inv = 1.0 / jnp.maximum(c0_ref[...] + c1_ref[...], 1.0)
    z = (jnp.dot(agg * inv, wl_ref[...], preferred_element_type=jnp.float32)
         + xw_ref[...])
    o_ref[...] = jnp.maximum(z, 0.0) if relu else z


def _tc_post(relu, out_rows, block_rows, acc0, acc1, cnt0, cnt1, xw, wlt):
    grid = out_rows // block_rows
    row_spec = pl.BlockSpec((block_rows, D), lambda i: (i, 0))
    cnt_spec = pl.BlockSpec((block_rows, 1), lambda i: (i, 0))
    full = pl.BlockSpec((D, D), lambda i: (0, 0))
    return pl.pallas_call(
        functools.partial(_tc_post_body, relu),
        grid=(grid,),
        in_specs=[row_spec, row_spec, cnt_spec, cnt_spec, row_spec, full],
        out_specs=row_spec,
        out_shape=jax.ShapeDtypeStruct((out_rows, D), jnp.float32),
    )(acc0, acc1, cnt0, cnt1, xw, wlt)


def kernel(x, edge_index, Wl1, bl1, Wr1, Wl2, bl2, Wr2):
    src = edge_index[0].astype(jnp.int32)
    dst = edge_index[1].astype(jnp.int32)
    pad = EPAD - E
    arange_pad = jnp.arange(pad, dtype=jnp.int32)
    src_pad = jnp.concatenate([src, arange_pad % N])
    dst_pad = jnp.concatenate([dst, N + arange_pad % (NPAD - N)])
    dst4 = dst_pad.reshape(NW, NB, B, K)

    xw1 = _tc_pre(x, Wr1.T, bl1.reshape(1, D))
    acc0, acc1, cnt0, cnt1 = _sc_agg_kernel(True)(x, src_pad, dst4)
    cnt0 = cnt0.reshape(NPAD, 1)
    cnt1 = cnt1.reshape(NPAD, 1)

    h = _tc_post(True, N, 2000, acc0, acc1, cnt0, cnt1, xw1, Wl1.T)

    hw2 = _tc_pre(h, Wr2.T, bl2.reshape(1, D))
    b0, b1 = _sc_agg_kernel(False)(h, src_pad, dst4)

    out = _tc_post(False, N, 2000, b0, b1, cnt0, cnt1, hw2, Wl2.T)
    return out
